# baseline (speedup 1.0000x reference)
import functools
import math

import jax
import jax.numpy as jnp
from jax import lax
from jax.experimental import pallas as pl
from jax.experimental.pallas import tpu as pltpu
from jax.experimental.pallas import tpu_sc as plsc

VOCAB = 1_000_000
D = 64
B = 16384
SCALE = math.sqrt(D)

NC = 2
NS = 16
NW = NC * NS
G = 16
WIN = 128
WPW = 244
LPW = WPW * WIN
TAIL_LO = 999_936
TAIL_W = VOCAB - TAIL_LO
NWIN_MAX = 249
BCAP = 8
NBUF = 4
PRE = 3
NSLOT = 16
NBANK = 2


def _body(idx_hbm, tableT_hbm, out_hbm,
          idx_all, lo_ids, lo_pos, wbuf, tailbuf, bkt_ids, bkt_pos,
          stag, poslist, wlist, tmp_ids, tmp_pos, ctr, wcnt,
          sem_in, sem_out):
    wid = lax.axis_index("s") * NC + lax.axis_index("c")
    is_last = wid == NW - 1
    lane_lo = wid * LPW
    lane_hi = jnp.where(is_last, VOCAB, lane_lo + LPW)
    full_w = jnp.where(is_last, NWIN_MAX - 1, WPW)
    ctr[0] = 0
    ctr[1] = 0
    ctr[2] = 0
    ctr[3] = 0

    def zw(i, carry):
        wcnt[i] = 0
        return carry

    lax.fori_loop(0, NWIN_MAX, zw, 0)

    pltpu.sync_copy(idx_hbm, idx_all)
    iota = lax.iota(jnp.int32, G)
    lane0 = iota == 0

    def splat(x):
        return jnp.full((G,), x, jnp.int32)

    def handle_tok(vj, pj):
        w = lax.shift_right_logical(vj - lane_lo, 7)
        c = wcnt[w]
        wcnt[w] = c + 1

        @pl.when(c < BCAP)
        def _():
            flat = w * BCAP + c
            plsc.store_scatter(bkt_ids, [splat(flat)], splat(vj), mask=lane0)
            plsc.store_scatter(bkt_pos, [splat(flat)], splat(pj), mask=lane0)

        @pl.when(c >= BCAP)
        def _():
            lidx = ctr[3]
            plsc.store_scatter(lo_ids, [splat(lidx)], splat(vj), mask=lane0)
            plsc.store_scatter(lo_pos, [splat(lidx)], splat(pj), mask=lane0)
            ctr[3] = lidx + 1

    def sel(g, carry):
        v = idx_all[pl.ds(g * G, G)]
        m = (v >= lane_lo) & (v < lane_hi)
        cnt = plsc.all_reduce_population_count(m)[0]

        @pl.when(cnt > 0)
        def _(v=v, m=m, g=g):
            plsc.store_compressed(tmp_ids.at[pl.ds(0, G)], v, mask=m)
            plsc.store_compressed(
                tmp_pos.at[pl.ds(0, G)], iota + g * G, mask=m
            )

            def each(j, c2):
                vj = tmp_ids[pl.ds(j, G)][0]
                pj = tmp_pos[pl.ds(j, G)][0]
                handle_tok(vj, pj)
                return c2

            lax.fori_loop(0, cnt, each, 0)

        return carry

    lax.fori_loop(0, B // G, sel, 0)

    def bw(w, nw):
        n = wcnt[w]
        hit = jnp.where((n > 0) & (w < full_w), 1, 0)

        @pl.when(hit > 0)
        def _(w=w, nw=nw):
            plsc.store_scatter(wlist, [splat(nw)], splat(w), mask=lane0)

        return nw + hit

    nwin = lax.fori_loop(0, NWIN_MAX, bw, 0)

    def do_token(gather_fn, l, pos):
        t = ctr[0]
        slot = jnp.bitwise_and(t, NSLOT - 1)
        bank = jnp.bitwise_and(lax.shift_right_logical(t, 4), NBANK - 1)

        @pl.when((slot == 0) & (t >= NBANK * NSLOT))
        def _():
            pltpu.make_async_copy(
                stag.at[0], out_hbm.at[poslist.at[0]], sem_out
            ).wait()
            ctr[1] = ctr[1] + 1

        lsplat = splat(l)
        for g3 in range(D // G):
            vals = gather_fn(iota + g3 * G, lsplat)
            stag[bank, slot, pl.ds(g3 * G, G)] = vals * SCALE
        plsc.store_scatter(
            poslist, [splat(bank), splat(slot)], splat(pos), mask=lane0
        )
        ctr[2] = pos

        @pl.when(slot == NSLOT - 1)
        def _():
            pltpu.async_copy(
                stag.at[bank], out_hbm.at[poslist.at[bank]], sem_out
            )

        ctr[0] = t + 1

    def scan_spill(gather_fn, c_lo, width):
        lcnt = ctr[3]
        ngl = (lcnt + G - 1) // G

        def grp(g2, carry):
            v = lo_ids[pl.ds(g2 * G, G)]
            p = lo_pos[pl.ds(g2 * G, G)]
            valid = iota < (lcnt - g2 * G)
            m = valid & (v >= c_lo) & (v < c_lo + width)
            cnt = plsc.all_reduce_population_count(m)[0]

            @pl.when(cnt > 0)
            def _():
                plsc.store_compressed(tmp_ids.at[pl.ds(0, G)], v, mask=m)
                plsc.store_compressed(tmp_pos.at[pl.ds(0, G)], p, mask=m)

                def each(j, c2):
                    vj = tmp_ids[pl.ds(j, G)][0]
                    pj = tmp_pos[pl.ds(j, G)][0]
                    do_token(gather_fn, vj - c_lo, pj)
                    return c2

                lax.fori_loop(0, cnt, each, 0)

            return carry

        lax.fori_loop(0, ngl, grp, 0)

    def process_window(gather_fn, wbase, w):
        nb = jnp.minimum(wcnt[w], BCAP)

        def each(j, carry):
            vj = bkt_ids[pl.ds(w * BCAP + j, G)][0]
            pj = bkt_pos[pl.ds(w * BCAP + j, G)][0]
            do_token(gather_fn, vj - wbase, pj)
            return carry

        lax.fori_loop(0, nb, each, 0)

    def start_win(i):
        w = wlist[pl.ds(i, G)][0]
        pltpu.async_copy(
            tableT_hbm.at[:, pl.ds(lane_lo + w * WIN, WIN)],
            wbuf.at[jnp.bitwise_and(i, NBUF - 1)],
            sem_in,
        )

    for k in range(PRE):
        @pl.when(k < nwin)
        def _(k=k):
            start_win(jnp.int32(k))

    def win_loop(i, carry):
        pltpu.make_async_copy(
            tableT_hbm.at[:, pl.ds(0, WIN)], wbuf.at[0], sem_in
        ).wait()
        w = wlist[pl.ds(i, G)][0]
        wbase = lane_lo + w * WIN
        bisplat = splat(jnp.bitwise_and(i, NBUF - 1))

        def gfn(rows, lanes):
            return plsc.load_gather(wbuf, [bisplat, rows, lanes])

        process_window(gfn, wbase, w)
        scan_spill(gfn, wbase, WIN)

        @pl.when(i + PRE < nwin)
        def _():
            start_win(i + PRE)

        return carry

    lax.fori_loop(0, nwin, win_loop, 0)

    @pl.when(is_last)
    def _():
        pltpu.sync_copy(tableT_hbm.at[:, pl.ds(TAIL_LO, TAIL_W)], tailbuf)

        def gfn(rows, lanes):
            return plsc.load_gather(tailbuf, [rows, lanes])

        process_window(gfn, TAIL_LO, NWIN_MAX - 1)
        scan_spill(gfn, TAIL_LO, TAIL_W)

    t = ctr[0]
    r = jnp.bitwise_and(t, NSLOT - 1)
    fbank = jnp.bitwise_and(lax.shift_right_logical(t, 4), NBANK - 1)

    @pl.when(r > 0)
    def _():
        lastpos = ctr[2]
        plsc.store_scatter(
            poslist, [splat(fbank), iota], splat(lastpos), mask=iota >= r
        )
        for j in range(NSLOT):
            @pl.when(j >= r)
            def _(j=j):
                for g3 in range(D // G):
                    sl = pl.ds(g3 * G, G)
                    stag[fbank, j, sl] = stag[fbank, r - 1, sl]

        pltpu.async_copy(
            stag.at[fbank], out_hbm.at[poslist.at[fbank]], sem_out
        )

    fired = t // NSLOT + jnp.where(r > 0, 1, 0)
    d = ctr[1]

    def drain_bank(i, carry):
        pltpu.make_async_copy(
            stag.at[0], out_hbm.at[poslist.at[0]], sem_out
        ).wait()
        return carry

    lax.fori_loop(0, fired - d, drain_bank, 0)


def kernel(token_ids, embedding_table):
    idx = token_ids.astype(jnp.int32)
    table_t = embedding_table.T
    run = functools.partial(
        pl.kernel,
        out_type=jax.ShapeDtypeStruct((B, 2 * D), jnp.float32),
        mesh=plsc.VectorSubcoreMesh(core_axis_name="c", subcore_axis_name="s"),
        compiler_params=pltpu.CompilerParams(needs_layout_passes=False),
        scratch_types=[
            pltpu.VMEM((B,), jnp.int32),
            pltpu.VMEM((B + G,), jnp.int32),
            pltpu.VMEM((B + G,), jnp.int32),
            pltpu.VMEM((NBUF, D, WIN), jnp.float32),
            pltpu.VMEM((D, TAIL_W), jnp.float32),
            pltpu.VMEM((NWIN_MAX * BCAP + G,), jnp.int32),
            pltpu.VMEM((NWIN_MAX * BCAP + G,), jnp.int32),
            pltpu.VMEM((NBANK, NSLOT, 2 * D), jnp.float32),
            pltpu.VMEM((NBANK, NSLOT), jnp.int32),
            pltpu.VMEM((NWIN_MAX + G,), jnp.int32),
            pltpu.VMEM((2 * G,), jnp.int32),
            pltpu.VMEM((2 * G,), jnp.int32),
            pltpu.SMEM((4,), jnp.int32),
            pltpu.SMEM((NWIN_MAX + 7,), jnp.int32),
            pltpu.SemaphoreType.DMA,
            pltpu.SemaphoreType.DMA,
        ],
    )(_body)
    return run(idx, table_t)[:, :D]


# speedup vs baseline: 1.1063x; 1.1063x over previous
import functools
import math

import jax
import jax.numpy as jnp
from jax import lax
from jax.experimental import pallas as pl
from jax.experimental.pallas import tpu as pltpu
from jax.experimental.pallas import tpu_sc as plsc

VOCAB = 1_000_000
D = 64
B = 16384
SCALE = math.sqrt(D)

NC = 2
NS = 16
NW = NC * NS
G = 16
WIN = 128
CHUNK_W = 512
WPW = 244
LPW = WPW * WIN
N_CHUNK = LPW // CHUNK_W
TAIL_LO = 999_936
NSLOT = 16
NBANK = 2


def _body(idx_hbm, tableT_hbm, out_hbm,
          idx_all, my_ids, my_pos, buf, tailbuf, stag, poslist,
          tmp_ids, tmp_pos, ctr, sem_in, sem_out):
    wid = lax.axis_index("s") * NC + lax.axis_index("c")
    is_last = wid == NW - 1
    lane_lo = wid * LPW
    lane_hi = jnp.where(is_last, VOCAB, lane_lo + LPW)
    ctr[0] = 0
    ctr[1] = 0

    pltpu.sync_copy(idx_hbm, idx_all)
    iota = lax.iota(jnp.int32, G)

    def start_chunk(c, slot):
        base = lane_lo + c * CHUNK_W
        for c0 in range(D // 8):
            pltpu.async_copy(
                tableT_hbm.at[pl.ds(8 * c0, 8), pl.ds(base, CHUNK_W)],
                buf.at[slot, pl.ds(8 * c0, 8)],
                sem_in,
            )

    start_chunk(0, 0)
    start_chunk(1, 1)

    def sel(g4, cur):
        for k in range(4):
            g = g4 * 4 + k
            v = idx_all[pl.ds(g * G, G)]
            m = (v >= lane_lo) & (v < lane_hi)
            cnt = plsc.all_reduce_population_count(m)[0]

            @pl.when(cnt > 0)
            def _(v=v, m=m, g=g, cur=cur):
                plsc.store_compressed(my_ids.at[pl.ds(cur, G)], v, mask=m)
                plsc.store_compressed(
                    my_pos.at[pl.ds(cur, G)], iota + g * G, mask=m
                )

            cur = cur + cnt
        return cur

    nmine = lax.fori_loop(0, B // G // 4, sel, 0)
    ngrp = (nmine + G - 1) // G

    lane0 = iota == 0

    def do_token(gather_fn, l, pos):
        t = ctr[0]
        slot = jnp.bitwise_and(t, NSLOT - 1)
        bank = jnp.bitwise_and(lax.shift_right_logical(t, 4), NBANK - 1)

        @pl.when((slot == 0) & (t >= NBANK * NSLOT))
        def _():
            pltpu.make_async_copy(
                stag.at[0], out_hbm.at[poslist.at[0]], sem_out
            ).wait()
            ctr[1] = ctr[1] + 1

        lsplat = jnp.full((G,), l, jnp.int32)
        for g3 in range(D // G):
            vals = gather_fn(iota + g3 * G, lsplat)
            stag[bank, slot, pl.ds(g3 * G, G)] = vals * SCALE
        plsc.store_scatter(
            poslist,
            [jnp.full((G,), bank, jnp.int32), jnp.full((G,), slot, jnp.int32)],
            jnp.full((G,), pos, jnp.int32),
            mask=lane0,
        )
        ctr[2] = pos

        @pl.when(slot == NSLOT - 1)
        def _():
            pltpu.async_copy(
                stag.at[bank], out_hbm.at[poslist.at[bank]], sem_out
            )

        ctr[0] = t + 1

    def scan_window(gather_fn, c_lo, width):
        def grp(g2, carry):
            v = my_ids[pl.ds(g2 * G, G)]
            p = my_pos[pl.ds(g2 * G, G)]
            valid = iota < (nmine - g2 * G)
            m = valid & (v >= c_lo) & (v < c_lo + width)
            cnt = plsc.all_reduce_population_count(m)[0]

            @pl.when(cnt > 0)
            def _():
                plsc.store_compressed(tmp_ids.at[pl.ds(0, G)], v, mask=m)
                plsc.store_compressed(tmp_pos.at[pl.ds(0, G)], p, mask=m)

                def each(j, c2):
                    vj = tmp_ids[pl.ds(j, G)][0]
                    pj = tmp_pos[pl.ds(j, G)][0]
                    do_token(gather_fn, vj - c_lo, pj)
                    return c2

                lax.fori_loop(0, cnt, each, 0)

            return carry

        lax.fori_loop(0, ngrp, grp, 0)

    trip = jnp.where(is_last, N_CHUNK + 1, N_CHUNK)

    def chunk_loop(c, carry):
        pltpu.make_async_copy(
            tableT_hbm.at[:, pl.ds(0, CHUNK_W)], buf.at[0], sem_in
        ).wait()
        cbsplat = jnp.full((G,), lax.rem(c, 2), jnp.int32)

        def gather_buf(rows, lanes):
            return plsc.load_gather(buf, [cbsplat, rows, lanes])

        scan_window(gather_buf, lane_lo + c * CHUNK_W, CHUNK_W)

        @pl.when(c + 2 < trip)
        def _():
            start_chunk(c + 2, lax.rem(c, 2))

        return carry

    lax.fori_loop(0, trip, chunk_loop, 0)

    @pl.when(is_last)
    def _():
        pltpu.sync_copy(
            tableT_hbm.at[:, pl.ds(TAIL_LO, VOCAB - TAIL_LO)], tailbuf
        )

        def gather_tail(rows, lanes):
            return plsc.load_gather(tailbuf, [rows, lanes])

        scan_window(gather_tail, TAIL_LO, VOCAB - TAIL_LO)

    t = ctr[0]
    r = jnp.bitwise_and(t, NSLOT - 1)
    fbank = jnp.bitwise_and(lax.shift_right_logical(t, 4), NBANK - 1)

    @pl.when(r > 0)
    def _():
        lastpos = ctr[2]
        plsc.store_scatter(
            poslist,
            [jnp.full((G,), fbank, jnp.int32), iota],
            jnp.full((G,), lastpos, jnp.int32),
            mask=iota >= r,
        )
        for j in range(NSLOT):
            @pl.when(j >= r)
            def _(j=j):
                for g3 in range(D // G):
                    sl = pl.ds(g3 * G, G)
                    stag[fbank, j, sl] = stag[fbank, r - 1, sl]

        pltpu.async_copy(
            stag.at[fbank], out_hbm.at[poslist.at[fbank]], sem_out
        )

    fired = t // NSLOT + jnp.where(r > 0, 1, 0)
    d = ctr[1]

    def drain_bank(i, carry):
        pltpu.make_async_copy(
            stag.at[0], out_hbm.at[poslist.at[0]], sem_out
        ).wait()
        return carry

    lax.fori_loop(0, fired - d, drain_bank, 0)


def kernel(token_ids, embedding_table):
    idx = token_ids.astype(jnp.int32)
    table_t = embedding_table.T
    run = functools.partial(
        pl.kernel,
        out_type=jax.ShapeDtypeStruct((B, 2 * D), jnp.float32),
        mesh=plsc.VectorSubcoreMesh(core_axis_name="c", subcore_axis_name="s"),
        compiler_params=pltpu.CompilerParams(needs_layout_passes=False),
        scratch_types=[
            pltpu.VMEM((B,), jnp.int32),
            pltpu.VMEM((B + G,), jnp.int32),
            pltpu.VMEM((B + G,), jnp.int32),
            pltpu.VMEM((2, D, CHUNK_W), jnp.float32),
            pltpu.VMEM((D, VOCAB - TAIL_LO), jnp.float32),
            pltpu.VMEM((NBANK, NSLOT, 2 * D), jnp.float32),
            pltpu.VMEM((NBANK, NSLOT), jnp.int32),
            pltpu.VMEM((2 * G,), jnp.int32),
            pltpu.VMEM((2 * G,), jnp.int32),
            pltpu.SMEM((4,), jnp.int32),
            pltpu.SemaphoreType.DMA,
            pltpu.SemaphoreType.DMA,
        ],
    )(_body)
    return run(idx, table_t)[:, :D]
